# baseline (speedup 1.0000x reference)
import functools

import jax
import jax.numpy as jnp
from jax import lax
from jax.experimental import pallas as pl
from jax.experimental.pallas import tpu as pltpu
from jax.experimental.pallas import tpu_sc as plsc

D_MODEL = 768
VOCAB = 100000
B = 4
S = 4096
EPS = 1e-12
NW = 32
P_PER_W = S // NW
G = 64
NJ = D_MODEL // 16


def _ln_chunk(rows_v, pos_v, g_v, b_v):
    def token_body(t, _):
        acc = jnp.zeros((16,), jnp.float32)
        acc2 = jnp.zeros((16,), jnp.float32)
        for j in range(NJ):
            sl = pl.ds(j * 16, 16)
            x = rows_v[t, sl] + pos_v[t, sl]
            rows_v[t, sl] = x
            acc = acc + x
            acc2 = acc2 + x * x
        s1 = jnp.sum(acc)
        s2 = jnp.sum(acc2)
        mean = s1 * (1.0 / D_MODEL)
        var = s2 * (1.0 / D_MODEL) - mean * mean
        ones = jnp.ones((16,), jnp.float32)
        mean_v = mean * ones
        v = (var + EPS) * ones
        i = plsc.bitcast(v, jnp.int32)
        i = 0x5F3759DF - (i >> 1)
        y = plsc.bitcast(i, jnp.float32)
        half_v = 0.5 * v
        for _n in range(3):
            y = y * (1.5 - half_v * y * y)
        for j in range(NJ):
            sl = pl.ds(j * 16, 16)
            x = rows_v[t, sl]
            rows_v[t, sl] = (x - mean_v) * y * g_v[sl] + b_v[sl]
        return 0

    lax.fori_loop(0, G, token_body, 0)


def _sc_body(ids_hbm, wt_hbm, pt_hbm, g_hbm, b_hbm, out_hbm,
             idx_v, rows_v, pos_v, g_v, b_v, sem):
    wid = lax.axis_index("s") * 2 + lax.axis_index("c")
    p0 = wid * P_PER_W
    pltpu.sync_copy(g_hbm, g_v)
    pltpu.sync_copy(b_hbm, b_v)

    def pc_body(pc, _):
        pbase = p0 + pc * G
        pltpu.sync_copy(pt_hbm.at[pl.ds(pbase, G)], pos_v)

        def b_body(bb, _):
            tok = bb * S + pbase
            pltpu.sync_copy(ids_hbm.at[pl.ds(tok, G)], idx_v)
            pltpu.async_copy(wt_hbm.at[idx_v], rows_v, sem).wait()
            _ln_chunk(rows_v, pos_v, g_v, b_v)
            pltpu.sync_copy(rows_v, out_hbm.at[pl.ds(tok, G)])
            return 0

        lax.fori_loop(0, B, b_body, 0)
        return 0

    lax.fori_loop(0, P_PER_W // G, pc_body, 0)


@jax.jit
def _run(ids_flat, word_table, pos_table, gamma, beta):
    mesh = plsc.VectorSubcoreMesh(core_axis_name="c", subcore_axis_name="s")
    k = pl.kernel(
        _sc_body,
        out_type=jax.ShapeDtypeStruct((B * S, D_MODEL), jnp.float32),
        mesh=mesh,
        compiler_params=pltpu.CompilerParams(needs_layout_passes=False),
        scratch_types=[
            pltpu.VMEM((G,), jnp.int32),
            pltpu.VMEM((G, D_MODEL), jnp.float32),
            pltpu.VMEM((G, D_MODEL), jnp.float32),
            pltpu.VMEM((D_MODEL,), jnp.float32),
            pltpu.VMEM((D_MODEL,), jnp.float32),
            pltpu.SemaphoreType.DMA,
        ],
    )
    return k(ids_flat, word_table, pos_table, gamma, beta)


def kernel(input_ids, word_table, pos_table, gamma, beta):
    ids_flat = jnp.reshape(input_ids.astype(jnp.int32), (B * S,))
    out = _run(ids_flat, word_table, pos_table, gamma, beta)
    return jnp.reshape(out, (B, S, D_MODEL))
